# baseline (speedup 1.0000x reference)
import functools

import jax
import jax.numpy as jnp
import numpy as np
from jax import lax
from jax.experimental import pallas as pl
from jax.experimental.pallas import tpu as pltpu
from jax.experimental.pallas import tpu_sc as plsc

_BETA = 0.2
_PERM_CACHE = {}


def _perm_indices(batch):
    key = batch
    if key not in _PERM_CACHE:
        cpu = jax.local_devices(backend="cpu")[0]
        with jax.default_device(cpu), jax.ensure_compile_time_eval():
            perm = np.asarray(jax.random.permutation(jax.random.key(1), batch))
        _PERM_CACHE[key] = perm.astype(np.int32)
    return _PERM_CACHE[key]


def _sc_gather(ind_col, pos_col, idx, nw, chunks, chunk):
    mesh = plsc.VectorSubcoreMesh(core_axis_name="c", subcore_axis_name="s")
    nc = 2

    @functools.partial(
        pl.kernel,
        mesh=mesh,
        out_type=[
            jax.ShapeDtypeStruct((nw, chunks, chunk), jnp.float32),
            jax.ShapeDtypeStruct((nw, chunks, chunk), jnp.float32),
        ],
        scratch_types=[
            pltpu.VMEM((chunks, chunk), jnp.int32),
            pltpu.VMEM((chunks, chunk), jnp.float32),
            pltpu.VMEM((chunks, chunk), jnp.float32),
            pltpu.SemaphoreType.DMA,
            pltpu.SemaphoreType.DMA,
        ],
    )
    def gather_kernel(ind_hbm, pos_hbm, idx_hbm, si_hbm, sp_hbm,
                      idx_v, a_v, b_v, sem_a, sem_b):
        wid = lax.axis_index("s") * nc + lax.axis_index("c")
        pltpu.sync_copy(idx_hbm.at[wid], idx_v)
        copies = []
        for j in range(chunks):
            copies.append(
                pltpu.async_copy(ind_hbm.at[idx_v.at[j]], a_v.at[j], sem_a))
            copies.append(
                pltpu.async_copy(pos_hbm.at[idx_v.at[j]], b_v.at[j], sem_b))
        for cp in copies:
            cp.wait()
        pltpu.sync_copy(a_v, si_hbm.at[wid])
        pltpu.sync_copy(b_v, sp_hbm.at[wid])

    return gather_kernel(ind_col, pos_col, idx)


def _tc_body(ind_ref, pos_ref, si_ref, sp_ref, out_ref):
    si = si_ref[...]
    sp = sp_ref[...]
    ind = ind_ref[...]
    pos = pos_ref[...]
    f = jnp.maximum(ind - si, 0.0)
    out_ref[...] = jnp.maximum(sp * f - pos * f + _BETA, 0.0) * f


def _tc_loss(indicator_vectors, positive, si, sp, bb):
    b, c = indicator_vectors.shape
    return pl.pallas_call(
        _tc_body,
        grid=(b // bb,),
        in_specs=[
            pl.BlockSpec((bb, c), lambda i: (i, 0)),
            pl.BlockSpec((bb, c), lambda i: (i, 0)),
            pl.BlockSpec((bb, 1), lambda i: (i, 0)),
            pl.BlockSpec((bb, 1), lambda i: (i, 0)),
        ],
        out_specs=pl.BlockSpec((bb, c), lambda i: (i, 0)),
        out_shape=jax.ShapeDtypeStruct((b, c), jnp.float32),
    )(indicator_vectors, positive, si, sp)


def _sc_loss(indicator_vectors, positive, si, sp):
    b, c = indicator_vectors.shape
    nw = 32
    rows_w = b // nw
    rgs = rows_w // 128
    jt = c // 128
    mesh = plsc.VectorSubcoreMesh(core_axis_name="c", subcore_axis_name="s")

    @functools.partial(
        pl.kernel,
        mesh=mesh,
        out_type=jax.ShapeDtypeStruct((b, c), jnp.float32),
        scratch_types=[
            pltpu.VMEM((128, 128), jnp.float32),
            pltpu.VMEM((128, 128), jnp.float32),
            pltpu.VMEM((128, 128), jnp.float32),
            pltpu.VMEM((128, 128), jnp.float32),
            pltpu.VMEM((128, 128), jnp.float32),
            pltpu.VMEM((128, 128), jnp.float32),
            pltpu.VMEM((rgs, 128), jnp.float32),
            pltpu.VMEM((rgs, 128), jnp.float32),
            pltpu.SemaphoreType.DMA,
            pltpu.SemaphoreType.DMA,
            pltpu.SemaphoreType.DMA,
            pltpu.SemaphoreType.DMA,
            pltpu.SemaphoreType.DMA,
            pltpu.SemaphoreType.DMA,
        ],
        compiler_params=pltpu.CompilerParams(
            use_tc_tiling_on_sc=True, needs_layout_passes=False),
    )
    def loss_kernel(ind_hbm, pos_hbm, si_hbm, sp_hbm, out_hbm,
                    ind_v0, ind_v1, pos_v0, pos_v1, out_v0, out_v1,
                    si_v, sp_v,
                    sem_a0, sem_a1, sem_b0, sem_b1, sem_o0, sem_o1):
        wid = lax.axis_index("s") * 2 + lax.axis_index("c")
        row_base = pl.multiple_of(wid * rows_w, 128)
        pltpu.sync_copy(si_hbm.at[wid], si_v)
        pltpu.sync_copy(sp_hbm.at[wid], sp_v)

        bufs = [
            (ind_v0, pos_v0, out_v0, sem_a0, sem_b0, sem_o0),
            (ind_v1, pos_v1, out_v1, sem_a1, sem_b1, sem_o1),
        ]

        def start_in(k):
            rg, j = divmod(k, jt)
            ind_b, pos_b, _, sa, sb, _ = bufs[k % 2]
            src_rows = pl.ds(row_base + rg * 128, 128)
            src_cols = pl.ds(j * 128, 128)
            return (
                pltpu.async_copy(ind_hbm.at[src_rows, src_cols], ind_b, sa),
                pltpu.async_copy(pos_hbm.at[src_rows, src_cols], pos_b, sb),
            )

        nslab = rgs * jt
        pend = {0: start_in(0)}
        out_cp = [None, None]
        for k in range(nslab):
            buf = k % 2
            if k + 1 < nslab:
                pend[k + 1] = start_in(k + 1)
            ca, cb = pend.pop(k)
            ca.wait()
            cb.wait()
            if out_cp[buf] is not None:
                out_cp[buf].wait()
            rg, j = divmod(k, jt)
            ind_b, pos_b, out_b, _, _, so = bufs[buf]

            def g_body(g, carry, ind_b=ind_b, pos_b=pos_b, out_b=out_b, rg=rg):
                siv = si_v[rg, pl.ds(g * 16, 16)]
                spv = sp_v[rg, pl.ds(g * 16, 16)]
                rowi = g * 16 + lax.iota(jnp.int32, 16)

                @plsc.parallel_loop(0, 128, 1, unroll=4)
                def c_body(cc):
                    coli = jnp.full((16,), cc, jnp.int32)
                    indv = plsc.load_gather(ind_b, [rowi, coli])
                    posv = plsc.load_gather(pos_b, [rowi, coli])
                    f = jnp.maximum(indv - siv, 0.0)
                    val = jnp.maximum((spv - posv) * f + _BETA, 0.0) * f
                    plsc.store_scatter(out_b, [rowi, coli], val)

                return carry

            lax.fori_loop(0, 128 // 16, g_body, 0)
            out_cp[buf] = pltpu.async_copy(
                out_b,
                out_hbm.at[pl.ds(row_base + rg * 128, 128),
                           pl.ds(j * 128, 128)],
                so)
        out_cp[0].wait()
        out_cp[1].wait()

    return loss_kernel(indicator_vectors, positive, si, sp)


def kernel(indicator_vectors, positive):
    b, c = positive.shape
    nw, chunk = 32, 128
    chunks = b // (nw * chunk)
    idx = jnp.asarray(_perm_indices(b).reshape(nw, chunks, chunk))
    si, sp = _sc_gather(
        indicator_vectors[:, 0], positive[:, 0], idx, nw, chunks, chunk)
    return _sc_loss(indicator_vectors, positive, si, sp)


# speedup vs baseline: 5.0368x; 5.0368x over previous
import functools

import jax
import jax.numpy as jnp
import numpy as np
from jax import lax
from jax.experimental import pallas as pl
from jax.experimental.pallas import tpu as pltpu
from jax.experimental.pallas import tpu_sc as plsc

_BETA = 0.2
_PERM_CACHE = {}


def _perm_indices(batch):
    key = batch
    if key not in _PERM_CACHE:
        cpu = jax.local_devices(backend="cpu")[0]
        with jax.default_device(cpu), jax.ensure_compile_time_eval():
            perm = np.asarray(jax.random.permutation(jax.random.key(1), batch))
        _PERM_CACHE[key] = perm.astype(np.int32)
    return _PERM_CACHE[key]


def _sc_gather(ind_col, pos_col, idx, nw, chunks, chunk):
    mesh = plsc.VectorSubcoreMesh(core_axis_name="c", subcore_axis_name="s")
    nc = 2

    @functools.partial(
        pl.kernel,
        mesh=mesh,
        out_type=[
            jax.ShapeDtypeStruct((nw, chunks, chunk), jnp.float32),
            jax.ShapeDtypeStruct((nw, chunks, chunk), jnp.float32),
        ],
        scratch_types=[
            pltpu.VMEM((chunks, chunk), jnp.int32),
            pltpu.VMEM((chunks, chunk), jnp.float32),
            pltpu.VMEM((chunks, chunk), jnp.float32),
            pltpu.SemaphoreType.DMA,
            pltpu.SemaphoreType.DMA,
        ],
    )
    def gather_kernel(ind_hbm, pos_hbm, idx_hbm, si_hbm, sp_hbm,
                      idx_v, a_v, b_v, sem_a, sem_b):
        wid = lax.axis_index("s") * nc + lax.axis_index("c")
        pltpu.sync_copy(idx_hbm.at[wid], idx_v)
        copies = []
        for j in range(chunks):
            copies.append(
                pltpu.async_copy(ind_hbm.at[idx_v.at[j]], a_v.at[j], sem_a))
            copies.append(
                pltpu.async_copy(pos_hbm.at[idx_v.at[j]], b_v.at[j], sem_b))
        for cp in copies:
            cp.wait()
        pltpu.sync_copy(a_v, si_hbm.at[wid])
        pltpu.sync_copy(b_v, sp_hbm.at[wid])

    return gather_kernel(ind_col, pos_col, idx)


def _tc_body(ind_ref, pos_ref, si_ref, sp_ref, out_ref):
    si = si_ref[...]
    sp = sp_ref[...]
    ind = ind_ref[...]
    pos = pos_ref[...]
    f = jnp.maximum(ind - si, 0.0)
    out_ref[...] = jnp.maximum(sp * f - pos * f + _BETA, 0.0) * f


def _tc_loss(indicator_vectors, positive, si, sp, bb):
    b, c = indicator_vectors.shape
    return pl.pallas_call(
        _tc_body,
        grid=(b // bb,),
        in_specs=[
            pl.BlockSpec((bb, c), lambda i: (i, 0)),
            pl.BlockSpec((bb, c), lambda i: (i, 0)),
            pl.BlockSpec((bb, 1), lambda i: (i, 0)),
            pl.BlockSpec((bb, 1), lambda i: (i, 0)),
        ],
        out_specs=pl.BlockSpec((bb, c), lambda i: (i, 0)),
        out_shape=jax.ShapeDtypeStruct((b, c), jnp.float32),
    )(indicator_vectors, positive, si, sp)


def _sc_loss(indicator_vectors, positive, si, sp):
    b, c = indicator_vectors.shape
    nw = 32
    rows_w = b // nw
    rgs = rows_w // 128
    jt = c // 128
    mesh = plsc.VectorSubcoreMesh(core_axis_name="c", subcore_axis_name="s")

    @functools.partial(
        pl.kernel,
        mesh=mesh,
        out_type=jax.ShapeDtypeStruct((b, c), jnp.float32),
        scratch_types=[
            pltpu.VMEM((128, 128), jnp.float32),
            pltpu.VMEM((128, 128), jnp.float32),
            pltpu.VMEM((128, 128), jnp.float32),
            pltpu.VMEM((128, 128), jnp.float32),
            pltpu.VMEM((128, 128), jnp.float32),
            pltpu.VMEM((128, 128), jnp.float32),
            pltpu.VMEM((rgs, 128), jnp.float32),
            pltpu.VMEM((rgs, 128), jnp.float32),
            pltpu.SemaphoreType.DMA,
            pltpu.SemaphoreType.DMA,
            pltpu.SemaphoreType.DMA,
            pltpu.SemaphoreType.DMA,
            pltpu.SemaphoreType.DMA,
            pltpu.SemaphoreType.DMA,
        ],
        compiler_params=pltpu.CompilerParams(
            use_tc_tiling_on_sc=True, needs_layout_passes=False),
    )
    def loss_kernel(ind_hbm, pos_hbm, si_hbm, sp_hbm, out_hbm,
                    ind_v0, ind_v1, pos_v0, pos_v1, out_v0, out_v1,
                    si_v, sp_v,
                    sem_a0, sem_a1, sem_b0, sem_b1, sem_o0, sem_o1):
        wid = lax.axis_index("s") * 2 + lax.axis_index("c")
        row_base = pl.multiple_of(wid * rows_w, 128)
        pltpu.sync_copy(si_hbm.at[wid], si_v)
        pltpu.sync_copy(sp_hbm.at[wid], sp_v)

        bufs = [
            (ind_v0, pos_v0, out_v0, sem_a0, sem_b0, sem_o0),
            (ind_v1, pos_v1, out_v1, sem_a1, sem_b1, sem_o1),
        ]

        nslab = rgs * jt

        def slab_window(kk):
            rg = kk // jt
            j = kk % jt
            r0 = pl.multiple_of(row_base + rg * 128, 128)
            c0 = pl.multiple_of(j * 128, 128)
            return pl.ds(r0, 128), pl.ds(c0, 128)

        def start_in(kk, bnum):
            ind_b, pos_b, _, sa, sb, _ = bufs[bnum]
            rows, cols = slab_window(kk)
            pltpu.async_copy(ind_hbm.at[rows, cols], ind_b, sa)
            pltpu.async_copy(pos_hbm.at[rows, cols], pos_b, sb)

        start_in(0, 0)
        start_in(1, 1)

        @pl.loop(0, nslab, step=2)
        def slab_pair(k):
            for bnum in range(2):
                kk = k + bnum
                ind_b, pos_b, out_b, sa, sb, so = bufs[bnum]
                rows, cols = slab_window(kk)
                rg = kk // jt
                pltpu.make_async_copy(ind_hbm.at[rows, cols], ind_b, sa).wait()
                pltpu.make_async_copy(pos_hbm.at[rows, cols], pos_b, sb).wait()

                @pl.when(kk >= 2)
                def _():
                    pltpu.make_async_copy(out_b, out_hbm.at[rows, cols],
                                          so).wait()

                @plsc.parallel_loop(0, 128 // 16, 1)
                def g_body(g):
                    siv = si_v[rg, pl.ds(g * 16, 16)]
                    spv = sp_v[rg, pl.ds(g * 16, 16)]
                    for r in range(16):
                        row = g * 16 + r
                        fs = jnp.full((16,), siv[r], jnp.float32)
                        fp = jnp.full((16,), spv[r], jnp.float32)
                        for cc in range(8):
                            sl = pl.ds(cc * 16, 16)
                            f = jnp.maximum(ind_b[row, sl] - fs, 0.0)
                            out_b[row, sl] = (
                                jnp.maximum((fp - pos_b[row, sl]) * f + _BETA,
                                            0.0) * f)

                pltpu.async_copy(out_b, out_hbm.at[rows, cols], so)

                @pl.when(kk + 2 < nslab)
                def _():
                    start_in(kk + 2, bnum)

        for bnum in range(2):
            _, _, out_b, _, _, so = bufs[bnum]
            rows, cols = slab_window(nslab - 2 + bnum)
            pltpu.make_async_copy(out_b, out_hbm.at[rows, cols], so).wait()

    return loss_kernel(indicator_vectors, positive, si, sp)


def kernel(indicator_vectors, positive):
    b, c = positive.shape
    nw, chunk = 32, 128
    chunks = b // (nw * chunk)
    idx = jnp.asarray(_perm_indices(b).reshape(nw, chunks, chunk))
    si, sp = _sc_gather(
        indicator_vectors[:, 0], positive[:, 0], idx, nw, chunks, chunk)
    return _sc_loss(indicator_vectors, positive, si, sp)
